# Initial kernel scaffold; baseline (speedup 1.0000x reference)
import jax
import jax.numpy as jnp
from jax.experimental import pallas as pl


def kernel(x, emb):
    raise NotImplementedError("write your pallas kernel here")


# speedup vs baseline: 5.7282x; 5.7282x over previous
import jax
import jax.numpy as jnp
from jax import lax
from jax.experimental import pallas as pl
from jax.experimental.pallas import tpu as pltpu
from jax.experimental.pallas import tpu_sc as plsc

NC, NS, L = 2, 16, 16
NW = NC * NS
CHUNK = 4096
EMB = 8


def _sc_body(x_hbm, emb_hbm, out_hbm, emb_v, idx_v, out_v):
    wid = lax.axis_index("s") * NC + lax.axis_index("c")
    n_total = x_hbm.shape[0]
    per_w = n_total // NW
    nchunks = per_w // CHUNK
    base = wid * per_w

    pltpu.sync_copy(emb_hbm, emb_v)
    iota8 = lax.iota(jnp.int32, L) * EMB

    def chunk_body(g, carry):
        start = base + g * CHUNK
        pltpu.sync_copy(x_hbm.at[pl.ds(start, CHUNK)], idx_v)

        def k_body(k, carry2):
            xv = idx_v[pl.ds(k * L, L)]
            xm = jnp.where((xv & 1) == 0, 0, xv)
            rb = xm * EMB
            obase = iota8 + k * (L * EMB)
            for j in range(EMB):
                v = plsc.load_gather(emb_v, [rb + j])
                plsc.store_scatter(out_v, [obase + j], v)
            return carry2

        lax.fori_loop(0, CHUNK // L, k_body, 0)
        pltpu.sync_copy(out_v, out_hbm.at[pl.ds(start * EMB, CHUNK * EMB)])
        return carry

    lax.fori_loop(0, nchunks, chunk_body, 0)


@jax.jit
def kernel(x, emb):
    B, S = x.shape
    V, D = emb.shape
    n = B * S
    xf = x.reshape(n).astype(jnp.int32)
    embf = emb.reshape(V * D).astype(jnp.float32)
    mesh = plsc.VectorSubcoreMesh(
        core_axis_name="c", subcore_axis_name="s", num_cores=NC, num_subcores=NS
    )
    out = pl.kernel(
        _sc_body,
        out_type=jax.ShapeDtypeStruct((n * D,), jnp.float32),
        mesh=mesh,
        scratch_types=[
            pltpu.VMEM((V * D,), jnp.float32),
            pltpu.VMEM((CHUNK,), jnp.int32),
            pltpu.VMEM((CHUNK * EMB,), jnp.float32),
        ],
        compiler_params=pltpu.CompilerParams(needs_layout_passes=False),
    )(xf, embf)
    return out.reshape(B, S, D)
